# Initial kernel scaffold; baseline (speedup 1.0000x reference)
import jax
import jax.numpy as jnp
from jax.experimental import pallas as pl


def kernel(x, table):
    raise NotImplementedError("write your pallas kernel here")


# speedup vs baseline: 1.6305x; 1.6305x over previous
import functools

import jax
import jax.numpy as jnp
from jax import lax
from jax.experimental import pallas as pl
from jax.experimental.pallas import tpu as pltpu
from jax.experimental.pallas import tpu_sc as plsc

B = 16384
V = 89
D = 64
L = 16
NC = 2
NS = 16
NW = NC * NS
BPW = B // NW

_mesh = plsc.VectorSubcoreMesh(core_axis_name="c", subcore_axis_name="s")


@functools.partial(
    pl.kernel,
    out_type=jax.ShapeDtypeStruct((D, B), jnp.float32),
    mesh=_mesh,
    compiler_params=pltpu.CompilerParams(needs_layout_passes=False),
    scratch_types=[
        pltpu.VMEM((BPW,), jnp.int32),
        pltpu.VMEM((V * D,), jnp.float32),
        pltpu.VMEM((D, BPW), jnp.float32),
    ],
)
def _emb_transpose(x_hbm, table_hbm, out_hbm, idx_v, tab_v, out_v):
    wid = lax.axis_index("s") * NC + lax.axis_index("c")
    base = wid * BPW
    pltpu.sync_copy(x_hbm.at[pl.ds(base, BPW)], idx_v)
    pltpu.sync_copy(table_hbm, tab_v)

    def group(j, carry):
        xv = idx_v[pl.ds(j * L, L)]
        addr = xv * D
        for d in range(D):
            out_v[d, pl.ds(j * L, L)] = plsc.load_gather(tab_v, [addr + d])
        return carry

    lax.fori_loop(0, BPW // L, group, 0)
    pltpu.sync_copy(out_v, out_hbm.at[:, pl.ds(base, BPW)])


def kernel(x, table):
    return _emb_transpose(x.astype(jnp.int32), table.reshape(V * D))


# speedup vs baseline: 2.0507x; 1.2577x over previous
import functools

import jax
import jax.numpy as jnp
from jax import lax
from jax.experimental import pallas as pl
from jax.experimental.pallas import tpu as pltpu
from jax.experimental.pallas import tpu_sc as plsc

B = 16384
V = 89
D = 64
L = 16
NC = 2
NS = 16
NW = NC * NS
BPW = B // NW

_mesh = plsc.VectorSubcoreMesh(core_axis_name="c", subcore_axis_name="s")


@functools.partial(
    pl.kernel,
    out_type=jax.ShapeDtypeStruct((D, B), jnp.float32),
    mesh=_mesh,
    compiler_params=pltpu.CompilerParams(needs_layout_passes=False),
    scratch_types=[
        pltpu.VMEM((BPW,), jnp.int32),
        pltpu.VMEM((V * D,), jnp.float32),
        pltpu.VMEM((D, BPW), jnp.float32),
    ],
)
def _emb_transpose(x_hbm, table_hbm, out_hbm, idx_v, tab_v, out_v):
    wid = lax.axis_index("s") * NC + lax.axis_index("c")
    base = wid * BPW
    pltpu.sync_copy(x_hbm.at[pl.ds(base, BPW)], idx_v)
    pltpu.sync_copy(table_hbm, tab_v)

    @plsc.parallel_loop(0, BPW, L)
    def group(b):
        xv = idx_v[pl.ds(b, L)]
        addr = xv * D
        for d in range(D):
            out_v[d, pl.ds(b, L)] = plsc.load_gather(tab_v, [addr + d])
    pltpu.sync_copy(out_v, out_hbm.at[:, pl.ds(base, BPW)])


def kernel(x, table):
    return _emb_transpose(x.astype(jnp.int32), table.reshape(V * D))
